# baseline (speedup 1.0000x reference)
import functools

import jax
import jax.numpy as jnp
from jax import lax
from jax.experimental import pallas as pl
from jax.experimental.pallas import tpu as pltpu
from jax.experimental.pallas import tpu_sc as plsc

N = 249216
C = 80
NX = 128
NY = 128
NBINS = NX * NY
NC = 2
NS = 16
NW = NC * NS
CHUNK = 128
NCH_HI = 61
W_HI = 27
ROWS_PER_TILE = NBINS // NS


CPP = 1
PROWS = CPP * CHUNK
NPH = 62


def _sc_body(x_hbm, ranks_hbm, out_hbm, xa, xb, ia, ib, sxa, sia, sxb, sib, grid):
    c = lax.axis_index("c")
    s = lax.axis_index("s")
    w = c * NS + s
    nch = jnp.where(w < W_HI, NCH_HI, NCH_HI - 1)
    cbase = jnp.where(w < W_HI, w * NCH_HI, w * (NCH_HI - 1) + W_HI)
    base = cbase * CHUNK

    def start_load(p, xbuf, ibuf, sx, si):
        pltpu.async_copy(x_hbm.at[pl.ds(base + p * PROWS, PROWS)], xbuf, sx)
        pltpu.async_copy(ranks_hbm.at[pl.ds(cbase + p * CPP, CPP)], ibuf, si)

    def wait_load(p, xbuf, ibuf, sx, si):
        pltpu.make_async_copy(x_hbm.at[pl.ds(base + p * PROWS, PROWS)], xbuf, sx).wait()
        pltpu.make_async_copy(ranks_hbm.at[pl.ds(cbase + p * CPP, CPP)], ibuf, si).wait()

    def scatter_phase(p, xbuf, ibuf):
        for k in range(CPP):
            @pl.when(CPP * p + k < nch)
            def _():
                pltpu.sync_copy(
                    xbuf.at[pl.ds(k * CHUNK, CHUNK)], grid.at[ibuf.at[k]], add=True
                )

    z16 = jnp.zeros((16,), jnp.float32)

    def zrow(i):
        for j in range(C // 16):
            xa[i, pl.ds(j * 16, 16)] = z16
        return 0

    lax.fori_loop(0, CHUNK, lambda i, _: zrow(i), 0)
    for i in range(ROWS_PER_TILE // CHUNK):
        pltpu.sync_copy(
            xa.at[pl.ds(0, CHUNK)],
            grid.at[pl.ds(s * ROWS_PER_TILE + i * CHUNK, CHUNK)],
        )
    plsc.subcore_barrier()

    start_load(0, xa, ia, sxa, sia)

    def pair_body(q, _):
        pa = 2 * q
        pb = 2 * q + 1

        @pl.when(CPP * pb < nch)
        def _():
            start_load(pb, xb, ib, sxb, sib)

        @pl.when(CPP * pa < nch)
        def _():
            wait_load(pa, xa, ia, sxa, sia)
            scatter_phase(pa, xa, ia)

        @pl.when(CPP * (pa + 2) < nch)
        def _():
            start_load(pa + 2, xa, ia, sxa, sia)

        @pl.when(CPP * pb < nch)
        def _():
            wait_load(pb, xb, ib, sxb, sib)
            scatter_phase(pb, xb, ib)

        return 0

    lax.fori_loop(0, NPH // 2, pair_body, 0)
    plsc.subcore_barrier()

    for i in range(ROWS_PER_TILE // CHUNK):
        r = s * ROWS_PER_TILE + i * CHUNK
        pltpu.sync_copy(
            grid.at[pl.ds(r, CHUNK)],
            out_hbm.at[pl.ds(c * NBINS + r, CHUNK)],
        )


_sc_scatter = functools.partial(
    pl.kernel,
    out_type=jax.ShapeDtypeStruct((NC * NBINS, C), jnp.float32),
    mesh=plsc.VectorSubcoreMesh(core_axis_name="c", subcore_axis_name="s"),
    compiler_params=pltpu.CompilerParams(use_tc_tiling_on_sc=False),
    scratch_types=[
        pltpu.VMEM((PROWS, C), jnp.float32),
        pltpu.VMEM((PROWS, C), jnp.float32),
        pltpu.VMEM((CPP, CHUNK), jnp.int32),
        pltpu.VMEM((CPP, CHUNK), jnp.int32),
        pltpu.SemaphoreType.DMA,
        pltpu.SemaphoreType.DMA,
        pltpu.SemaphoreType.DMA,
        pltpu.SemaphoreType.DMA,
        pltpu.VMEM_SHARED((NBINS, C), jnp.float32),
    ],
)(_sc_body)


NCHUNKS = N // CHUNK
_RBLK = 128
_RGRID = -(-NCHUNKS // _RBLK)


def _ranks_body(g_ref, o_ref):
    gf = g_ref[...].astype(jnp.float32)
    ii = lax.broadcasted_iota(jnp.int32, (2 * CHUNK, CHUNK), 0)
    jj = lax.broadcasted_iota(jnp.int32, (2 * CHUNK, CHUNK), 1)
    sel = jnp.where(
        ii == 2 * jj,
        jnp.float32(NY),
        jnp.where(ii == 2 * jj + 1, jnp.float32(1), jnp.float32(0)),
    )
    r = lax.dot_general(
        gf, sel, (((1,), (0,)), ((), ())), preferred_element_type=jnp.float32
    )
    o_ref[...] = r.astype(jnp.int32)


def _ranks(geom2):
    return pl.pallas_call(
        _ranks_body,
        grid=(_RGRID,),
        in_specs=[pl.BlockSpec((_RBLK, 2 * CHUNK), lambda i: (i, 0))],
        out_specs=pl.BlockSpec((_RBLK, CHUNK), lambda i: (i, 0)),
        out_shape=jax.ShapeDtypeStruct((NCHUNKS, CHUNK), jnp.int32),
    )(geom2)


_MBLK = 1024


def _merge_body(p_ref, o_ref):
    s = p_ref[0] + p_ref[1]
    eye = jnp.eye(_MBLK, dtype=jnp.float32)
    o_ref[...] = lax.dot_general(
        s, eye, (((0,), (0,)), ((), ())), preferred_element_type=jnp.float32
    )


def _merge(partials):
    return pl.pallas_call(
        _merge_body,
        grid=(NBINS // _MBLK,),
        in_specs=[pl.BlockSpec((NC, _MBLK, C), lambda i: (0, i, 0))],
        out_specs=pl.BlockSpec((C, _MBLK), lambda i: (0, i)),
        out_shape=jax.ShapeDtypeStruct((C, NBINS), jnp.float32),
    )(partials.reshape(NC, NBINS, C))


@jax.jit
def kernel(x, geom_xy):
    ranks = _ranks(geom_xy.reshape(NCHUNKS, 2 * CHUNK))
    partials = _sc_scatter(x, ranks)
    merged = _merge(partials)
    return merged.reshape(1, C, NX, NY)


# speedup vs baseline: 1.4156x; 1.4156x over previous
import functools

import jax
import jax.numpy as jnp
from jax import lax
from jax.experimental import pallas as pl
from jax.experimental.pallas import tpu as pltpu
from jax.experimental.pallas import tpu_sc as plsc

N = 249216
C = 80
NX = 128
NY = 128
NBINS = NX * NY
NC = 2
NS = 16
NW = NC * NS
CHUNK = 128
NCH_HI = 61
W_HI = 27
ROWS_PER_TILE = NBINS // NS


CPP = 1
PROWS = CPP * CHUNK
NPH = 62


def _sc_body(x_hbm, ranks_hbm, out_hbm, xa, xb, ia, ib, sxa, sia, sxb, sib, grid):
    c = lax.axis_index("c")
    s = lax.axis_index("s")
    w = c * NS + s
    nch = jnp.where(w < W_HI, NCH_HI, NCH_HI - 1)
    cbase = jnp.where(w < W_HI, w * NCH_HI, w * (NCH_HI - 1) + W_HI)
    base = cbase * CHUNK

    def start_load(p, xbuf, ibuf, sx, si):
        pltpu.async_copy(x_hbm.at[pl.ds(base + p * PROWS, PROWS)], xbuf, sx)
        pltpu.async_copy(ranks_hbm.at[pl.ds(cbase + p * CPP, CPP)], ibuf, si)

    def wait_load(p, xbuf, ibuf, sx, si):
        pltpu.make_async_copy(x_hbm.at[pl.ds(base + p * PROWS, PROWS)], xbuf, sx).wait()
        pltpu.make_async_copy(ranks_hbm.at[pl.ds(cbase + p * CPP, CPP)], ibuf, si).wait()

    def scatter_phase(p, xbuf, ibuf):
        for k in range(CPP):
            @pl.when(CPP * p + k < nch)
            def _():
                pltpu.sync_copy(
                    xbuf.at[pl.ds(k * CHUNK, CHUNK)], grid.at[ibuf.at[k]], add=True
                )

    z16 = jnp.zeros((16,), jnp.float32)

    def zrow(i):
        for j in range(C // 16):
            xa[i, pl.ds(j * 16, 16)] = z16
        return 0

    lax.fori_loop(0, CHUNK, lambda i, _: zrow(i), 0)
    for i in range(ROWS_PER_TILE // CHUNK):
        pltpu.sync_copy(
            xa.at[pl.ds(0, CHUNK)],
            grid.at[pl.ds(s * ROWS_PER_TILE + i * CHUNK, CHUNK)],
        )
    plsc.subcore_barrier()

    start_load(0, xa, ia, sxa, sia)

    def pair_body(q, _):
        pa = 2 * q
        pb = 2 * q + 1

        @pl.when(CPP * pb < nch)
        def _():
            start_load(pb, xb, ib, sxb, sib)

        @pl.when(CPP * pa < nch)
        def _():
            wait_load(pa, xa, ia, sxa, sia)
            scatter_phase(pa, xa, ia)

        @pl.when(CPP * (pa + 2) < nch)
        def _():
            start_load(pa + 2, xa, ia, sxa, sia)

        @pl.when(CPP * pb < nch)
        def _():
            wait_load(pb, xb, ib, sxb, sib)
            scatter_phase(pb, xb, ib)

        return 0

    lax.fori_loop(0, NPH // 2, pair_body, 0)
    plsc.subcore_barrier()

    for i in range(ROWS_PER_TILE // CHUNK):
        r = s * ROWS_PER_TILE + i * CHUNK
        pltpu.sync_copy(
            grid.at[pl.ds(r, CHUNK)],
            out_hbm.at[pl.ds(c * NBINS + r, CHUNK)],
        )


_sc_scatter = functools.partial(
    pl.kernel,
    out_type=jax.ShapeDtypeStruct((NC * NBINS, C), jnp.float32),
    mesh=plsc.VectorSubcoreMesh(core_axis_name="c", subcore_axis_name="s"),
    compiler_params=pltpu.CompilerParams(use_tc_tiling_on_sc=False),
    scratch_types=[
        pltpu.VMEM((PROWS, C), jnp.float32),
        pltpu.VMEM((PROWS, C), jnp.float32),
        pltpu.VMEM((CPP, CHUNK), jnp.int32),
        pltpu.VMEM((CPP, CHUNK), jnp.int32),
        pltpu.SemaphoreType.DMA,
        pltpu.SemaphoreType.DMA,
        pltpu.SemaphoreType.DMA,
        pltpu.SemaphoreType.DMA,
        pltpu.VMEM_SHARED((NBINS, C), jnp.float32),
    ],
)(_sc_body)


NCHUNKS = N // CHUNK
_RBLK = 128
_RGRID = -(-NCHUNKS // _RBLK)


def _ranks_body(g_ref, o_ref):
    gf = g_ref[...].astype(jnp.float32)
    ii = lax.broadcasted_iota(jnp.int32, (2 * CHUNK, CHUNK), 0)
    jj = lax.broadcasted_iota(jnp.int32, (2 * CHUNK, CHUNK), 1)
    sel = jnp.where(
        ii == 2 * jj,
        jnp.float32(NY),
        jnp.where(ii == 2 * jj + 1, jnp.float32(1), jnp.float32(0)),
    )
    r = lax.dot_general(
        gf, sel, (((1,), (0,)), ((), ())), preferred_element_type=jnp.float32
    )
    o_ref[...] = r.astype(jnp.int32)


def _ranks(geom2):
    return pl.pallas_call(
        _ranks_body,
        grid=(_RGRID,),
        in_specs=[pl.BlockSpec((_RBLK, 2 * CHUNK), lambda i: (i, 0))],
        out_specs=pl.BlockSpec((_RBLK, CHUNK), lambda i: (i, 0)),
        out_shape=jax.ShapeDtypeStruct((NCHUNKS, CHUNK), jnp.int32),
    )(geom2)


_MBLK = 1024


def _merge_body(p_ref, o_ref):
    s = p_ref[0] + p_ref[1]
    eye = jnp.eye(_MBLK, dtype=jnp.float32)
    o_ref[...] = lax.dot_general(
        s, eye, (((0,), (0,)), ((), ())), preferred_element_type=jnp.float32
    )


def _merge(partials):
    return pl.pallas_call(
        _merge_body,
        grid=(NBINS // _MBLK,),
        in_specs=[pl.BlockSpec((NC, _MBLK, C), lambda i: (0, i, 0))],
        out_specs=pl.BlockSpec((C, _MBLK), lambda i: (0, i)),
        out_shape=jax.ShapeDtypeStruct((C, NBINS), jnp.float32),
    )(partials.reshape(NC, NBINS, C))


@jax.jit
def kernel(x, geom_xy):
    ranks = (geom_xy * jnp.array([NY, 1], jnp.int32)).sum(axis=1)
    ranks = ranks.reshape(NCHUNKS, CHUNK)
    partials = _sc_scatter(x, ranks)
    merged = _merge(partials)
    return merged.reshape(1, C, NX, NY)
